# Initial kernel scaffold; baseline (speedup 1.0000x reference)
import jax
import jax.numpy as jnp
from jax.experimental import pallas as pl


def kernel(nucleotide_sequences, cigar_encodings, base_qualities, strand_flags, mate_pair_flags, nuc_table, cigar_table, quality_table, strand_table, mate_table, gate_W, gate_b, feat_W, feat_b):
    raise NotImplementedError("write your pallas kernel here")


# speedup vs baseline: 24.9354x; 24.9354x over previous
import functools

import jax
import jax.numpy as jnp
from jax import lax
from jax.experimental import pallas as pl
from jax.experimental.pallas import tpu as pltpu
from jax.experimental.pallas import tpu_sc as plsc

B, S, D = 4096, 200, 128
N = B * S
NCOMBO = 7 * 41 * 16
NROWS = 17 * NCOMBO
NW = 32
PER_W = N // NW
CHUNK = 128
NCHUNK = PER_W // CHUNK

IDX_R, IDX_C = 3200, 256
IDX_BR = 400


def _gated_body(small_ref, gw_ref, gb_ref, fw_ref, fb_ref, out_ref):
    small = small_ref[...]
    tg = jnp.dot(small, gw_ref[...], preferred_element_type=jnp.float32)
    tf = jnp.dot(small, fw_ref[...], preferred_element_type=jnp.float32)
    r = lax.broadcasted_iota(jnp.int32, (NCOMBO, 64), 0)
    col = lax.broadcasted_iota(jnp.int32, (NCOMBO, 64), 1)
    sel = (
        (col == r // 656)
        | (col == 7 + (r // 16) % 41)
        | (col == 48 + (r % 16) // 4)
        | (col == 52 + r % 4)
    )
    e = sel.astype(jnp.float32)
    g = jnp.dot(e, tg, preferred_element_type=jnp.float32) + gb_ref[...]
    f = jnp.dot(e, tf, preferred_element_type=jnp.float32) + fb_ref[...]
    out_ref[...] = g * (1.0 / (1.0 + jnp.exp(-g))) * f


def _expand_body(gated_ref, nuc_ref, out_ref):
    out_ref[...] = gated_ref[...] + nuc_ref[0]


def _index_body(n_ref, c_ref, q_ref, s_ref, m_ref, out_ref):
    q = jnp.clip(q_ref[...], 0, 40)
    combo = (c_ref[...] * 41 + q) * 16 + s_ref[...] * 4 + m_ref[...]
    out_ref[...] = n_ref[...] * NCOMBO + combo


_SC_MESH = plsc.VectorSubcoreMesh(core_axis_name="c", subcore_axis_name="s")


@functools.partial(
    pl.kernel,
    mesh=_SC_MESH,
    out_type=jax.ShapeDtypeStruct((N, D), jnp.float32),
    scratch_types=[
        pltpu.VMEM((CHUNK,), jnp.int32),
        pltpu.VMEM((CHUNK, D), jnp.float32),
        pltpu.SemaphoreType.DMA,
    ],
)
def _gather_k(table_hbm, idx_hbm, out_hbm, idx_v, rows_v, sem):
    wid = lax.axis_index("s") * 2 + lax.axis_index("c")
    base = wid * PER_W

    @pl.loop(0, NCHUNK)
    def _(j):
        off = base + j * CHUNK
        pltpu.sync_copy(idx_hbm.at[pl.ds(off, CHUNK)], idx_v)
        pltpu.async_copy(table_hbm.at[idx_v], rows_v, sem).wait()
        pltpu.sync_copy(rows_v, out_hbm.at[pl.ds(off, CHUNK)])


def kernel(nucleotide_sequences, cigar_encodings, base_qualities, strand_flags,
           mate_pair_flags, nuc_table, cigar_table, quality_table, strand_table,
           mate_table, gate_W, gate_b, feat_W, feat_b):
    f32 = jnp.float32

    small = jnp.zeros((64, 128), f32)
    small = small.at[0:7, 0:32].set(cigar_table)
    small = small.at[7:48, 32:64].set(quality_table[0:41])
    small = small.at[48:52, 64:96].set(strand_table)
    small = small.at[52:54, 96:128].set(mate_table[0:2])
    small = small.at[55:56, 96:128].set(mate_table[3:4])

    gated = pl.pallas_call(
        _gated_body,
        out_shape=jax.ShapeDtypeStruct((NCOMBO, 128), f32),
    )(small, gate_W, gate_b.reshape(1, 128), feat_W, feat_b.reshape(1, 128))

    tfull = pl.pallas_call(
        _expand_body,
        grid=(17,),
        in_specs=[
            pl.BlockSpec((NCOMBO, 128), lambda i: (0, 0)),
            pl.BlockSpec((1, 1, 128), lambda i: (i, 0, 0)),
        ],
        out_specs=pl.BlockSpec((NCOMBO, 128), lambda i: (i, 0)),
        out_shape=jax.ShapeDtypeStruct((NROWS, 128), f32),
    )(gated, nuc_table.reshape(17, 1, 128))

    shp = (IDX_R, IDX_C)
    idx_args = [a.reshape(shp) for a in (nucleotide_sequences, cigar_encodings,
                                         base_qualities, strand_flags,
                                         mate_pair_flags)]
    ridx = pl.pallas_call(
        _index_body,
        grid=(IDX_R // IDX_BR,),
        in_specs=[pl.BlockSpec((IDX_BR, IDX_C), lambda i: (i, 0))] * 5,
        out_specs=pl.BlockSpec((IDX_BR, IDX_C), lambda i: (i, 0)),
        out_shape=jax.ShapeDtypeStruct(shp, jnp.int32),
    )(*idx_args)

    out = _gather_k(tfull, ridx.reshape(N))
    return out.reshape(B, S, D)


# speedup vs baseline: 34.1252x; 1.3685x over previous
import functools

import jax
import jax.numpy as jnp
from jax import lax
from jax.experimental import pallas as pl
from jax.experimental.pallas import tpu as pltpu
from jax.experimental.pallas import tpu_sc as plsc

B, S, D = 4096, 200, 128
N = B * S
NCOMBO = 7 * 41 * 16
NROWS = 17 * NCOMBO
NW = 32
PER_W = N // NW
CHUNK = 128
NCHUNK = PER_W // CHUNK

IDX_R, IDX_C = 3200, 256
IDX_BR = 400


def _gated_body(small_ref, gw_ref, gb_ref, fw_ref, fb_ref, out_ref):
    small = small_ref[...]
    tg = jnp.dot(small, gw_ref[...], preferred_element_type=jnp.float32)
    tf = jnp.dot(small, fw_ref[...], preferred_element_type=jnp.float32)
    r = lax.broadcasted_iota(jnp.int32, (NCOMBO, 64), 0)
    col = lax.broadcasted_iota(jnp.int32, (NCOMBO, 64), 1)
    sel = (
        (col == r // 656)
        | (col == 7 + (r // 16) % 41)
        | (col == 48 + (r % 16) // 4)
        | (col == 52 + r % 4)
    )
    e = sel.astype(jnp.float32)
    g = jnp.dot(e, tg, preferred_element_type=jnp.float32) + gb_ref[...]
    f = jnp.dot(e, tf, preferred_element_type=jnp.float32) + fb_ref[...]
    out_ref[...] = g * (1.0 / (1.0 + jnp.exp(-g))) * f


def _expand_body(gated_ref, nuc_ref, out_ref):
    out_ref[...] = gated_ref[...] + nuc_ref[0]


def _index_body(n_ref, c_ref, q_ref, s_ref, m_ref, out_ref):
    q = jnp.clip(q_ref[...], 0, 40)
    combo = (c_ref[...] * 41 + q) * 16 + s_ref[...] * 4 + m_ref[...]
    out_ref[...] = n_ref[...] * NCOMBO + combo


_SC_MESH = plsc.VectorSubcoreMesh(core_axis_name="c", subcore_axis_name="s")

NBUF = 4


@functools.partial(
    pl.kernel,
    mesh=_SC_MESH,
    out_type=jax.ShapeDtypeStruct((N, D), jnp.float32),
    scratch_types=[
        pltpu.VMEM((NBUF, CHUNK), jnp.int32),
        pltpu.VMEM((NBUF, CHUNK, D), jnp.float32),
        pltpu.SemaphoreType.DMA((NBUF,)),
        pltpu.SemaphoreType.DMA,
        pltpu.SemaphoreType.DMA((NBUF,)),
    ],
)
def _gather_k(table_hbm, idx_hbm, out_hbm, idx_v, rows_v, si, sg, sw):
    wid = lax.axis_index("s") * 2 + lax.axis_index("c")
    base = wid * PER_W

    for b in range(NBUF):
        pltpu.async_copy(idx_hbm.at[pl.ds(base + b * CHUNK, CHUNK)],
                         idx_v.at[b], si.at[b])

    @pl.loop(0, NCHUNK, step=NBUF)
    def _(g):
        for b in range(NBUF):
            m = g + b
            off = base + m * CHUNK
            pltpu.make_async_copy(idx_hbm.at[pl.ds(off, CHUNK)],
                                  idx_v.at[b], si.at[b]).wait()

            @pl.when(m >= NBUF)
            def _():
                pltpu.make_async_copy(rows_v.at[b],
                                      out_hbm.at[pl.ds(base, CHUNK)],
                                      sw.at[b]).wait()

            pltpu.async_copy(table_hbm.at[idx_v.at[b]], rows_v.at[b],
                             sg).wait()

            @pl.when(m + NBUF < NCHUNK)
            def _():
                pltpu.async_copy(
                    idx_hbm.at[pl.ds(off + NBUF * CHUNK, CHUNK)],
                    idx_v.at[b], si.at[b])

            pltpu.async_copy(rows_v.at[b], out_hbm.at[pl.ds(off, CHUNK)],
                             sw.at[b])

    for b in range(NBUF):
        pltpu.make_async_copy(rows_v.at[b], out_hbm.at[pl.ds(base, CHUNK)],
                              sw.at[b]).wait()


def kernel(nucleotide_sequences, cigar_encodings, base_qualities, strand_flags,
           mate_pair_flags, nuc_table, cigar_table, quality_table, strand_table,
           mate_table, gate_W, gate_b, feat_W, feat_b):
    f32 = jnp.float32

    small = jnp.zeros((64, 128), f32)
    small = small.at[0:7, 0:32].set(cigar_table)
    small = small.at[7:48, 32:64].set(quality_table[0:41])
    small = small.at[48:52, 64:96].set(strand_table)
    small = small.at[52:54, 96:128].set(mate_table[0:2])
    small = small.at[55:56, 96:128].set(mate_table[3:4])

    gated = pl.pallas_call(
        _gated_body,
        out_shape=jax.ShapeDtypeStruct((NCOMBO, 128), f32),
    )(small, gate_W, gate_b.reshape(1, 128), feat_W, feat_b.reshape(1, 128))

    tfull = pl.pallas_call(
        _expand_body,
        grid=(17,),
        in_specs=[
            pl.BlockSpec((NCOMBO, 128), lambda i: (0, 0)),
            pl.BlockSpec((1, 1, 128), lambda i: (i, 0, 0)),
        ],
        out_specs=pl.BlockSpec((NCOMBO, 128), lambda i: (i, 0)),
        out_shape=jax.ShapeDtypeStruct((NROWS, 128), f32),
    )(gated, nuc_table.reshape(17, 1, 128))

    shp = (IDX_R, IDX_C)
    idx_args = [a.reshape(shp) for a in (nucleotide_sequences, cigar_encodings,
                                         base_qualities, strand_flags,
                                         mate_pair_flags)]
    ridx = pl.pallas_call(
        _index_body,
        grid=(IDX_R // IDX_BR,),
        in_specs=[pl.BlockSpec((IDX_BR, IDX_C), lambda i: (i, 0))] * 5,
        out_specs=pl.BlockSpec((IDX_BR, IDX_C), lambda i: (i, 0)),
        out_shape=jax.ShapeDtypeStruct(shp, jnp.int32),
    )(*idx_args)

    out = _gather_k(tfull, ridx.reshape(N))
    return out.reshape(B, S, D)


# speedup vs baseline: 40.8537x; 1.1972x over previous
import functools

import jax
import jax.numpy as jnp
from jax import lax
from jax.experimental import pallas as pl
from jax.experimental.pallas import tpu as pltpu
from jax.experimental.pallas import tpu_sc as plsc

B, S, D = 4096, 200, 128
N = B * S
NCOMBO = 7 * 41 * 16
NROWS = 17 * NCOMBO
NW = 32
PER_W = N // NW
CHUNK = 128
NCHUNK = PER_W // CHUNK

IDX_R, IDX_C = 3200, 256
IDX_BR = 400


def _gated_body(small_ref, gw_ref, gb_ref, fw_ref, fb_ref, out_ref):
    small = small_ref[...]
    tg = jnp.dot(small, gw_ref[...], preferred_element_type=jnp.float32)
    tf = jnp.dot(small, fw_ref[...], preferred_element_type=jnp.float32)
    r = lax.broadcasted_iota(jnp.int32, (NCOMBO, 64), 0)
    col = lax.broadcasted_iota(jnp.int32, (NCOMBO, 64), 1)
    sel = (
        (col == r // 656)
        | (col == 7 + (r // 16) % 41)
        | (col == 48 + (r % 16) // 4)
        | (col == 52 + r % 4)
    )
    e = sel.astype(jnp.float32)
    g = jnp.dot(e, tg, preferred_element_type=jnp.float32) + gb_ref[...]
    f = jnp.dot(e, tf, preferred_element_type=jnp.float32) + fb_ref[...]
    out_ref[...] = g * (1.0 / (1.0 + jnp.exp(-g))) * f


def _expand_body(gated_ref, nuc_ref, out_ref):
    out_ref[...] = gated_ref[...] + nuc_ref[0]


def _index_body(n_ref, c_ref, q_ref, s_ref, m_ref, out_ref):
    q = jnp.clip(q_ref[...], 0, 40)
    combo = (c_ref[...] * 41 + q) * 16 + s_ref[...] * 4 + m_ref[...]
    out_ref[...] = n_ref[...] * NCOMBO + combo


_SC_MESH = plsc.VectorSubcoreMesh(core_axis_name="c", subcore_axis_name="s")

NBUF = 4


@functools.partial(
    pl.kernel,
    mesh=_SC_MESH,
    out_type=jax.ShapeDtypeStruct((N, D), jnp.float32),
    scratch_types=[
        pltpu.VMEM((NBUF, CHUNK), jnp.int32),
        pltpu.VMEM((NBUF, CHUNK, D), jnp.float32),
        pltpu.SemaphoreType.DMA((NBUF,)),
        pltpu.SemaphoreType.DMA((NBUF,)),
        pltpu.SemaphoreType.DMA((NBUF,)),
    ],
)
def _gather_k(table_hbm, idx_hbm, out_hbm, idx_v, rows_v, si, sg, sw):
    wid = lax.axis_index("s") * 2 + lax.axis_index("c")
    base = wid * PER_W

    for b in range(NBUF):
        pltpu.async_copy(idx_hbm.at[pl.ds(base + b * CHUNK, CHUNK)],
                         idx_v.at[b], si.at[b])

    @pl.loop(0, NCHUNK, step=NBUF)
    def _(g):
        for b in range(NBUF):
            m = g + b
            off = base + m * CHUNK
            bp = (b - 1) % NBUF
            pltpu.make_async_copy(idx_hbm.at[pl.ds(off, CHUNK)],
                                  idx_v.at[b], si.at[b]).wait()

            @pl.when(m >= NBUF)
            def _():
                pltpu.make_async_copy(rows_v.at[b],
                                      out_hbm.at[pl.ds(base, CHUNK)],
                                      sw.at[b]).wait()

            pltpu.async_copy(table_hbm.at[idx_v.at[b]], rows_v.at[b],
                             sg.at[b])

            @pl.when(m >= 1)
            def _():
                pltpu.make_async_copy(table_hbm.at[idx_v.at[bp]],
                                      rows_v.at[bp], sg.at[bp]).wait()
                pltpu.async_copy(rows_v.at[bp],
                                 out_hbm.at[pl.ds(off - CHUNK, CHUNK)],
                                 sw.at[bp])

            @pl.when((m >= 1) & (m + NBUF - 1 < NCHUNK))
            def _():
                pltpu.async_copy(
                    idx_hbm.at[pl.ds(off + (NBUF - 1) * CHUNK, CHUNK)],
                    idx_v.at[bp], si.at[bp])

    last = NBUF - 1
    pltpu.make_async_copy(table_hbm.at[idx_v.at[last]], rows_v.at[last],
                          sg.at[last]).wait()
    pltpu.async_copy(rows_v.at[last],
                     out_hbm.at[pl.ds(base + (NCHUNK - 1) * CHUNK, CHUNK)],
                     sw.at[last])
    for b in range(NBUF):
        pltpu.make_async_copy(rows_v.at[b], out_hbm.at[pl.ds(base, CHUNK)],
                              sw.at[b]).wait()


def kernel(nucleotide_sequences, cigar_encodings, base_qualities, strand_flags,
           mate_pair_flags, nuc_table, cigar_table, quality_table, strand_table,
           mate_table, gate_W, gate_b, feat_W, feat_b):
    f32 = jnp.float32

    small = jnp.zeros((64, 128), f32)
    small = small.at[0:7, 0:32].set(cigar_table)
    small = small.at[7:48, 32:64].set(quality_table[0:41])
    small = small.at[48:52, 64:96].set(strand_table)
    small = small.at[52:54, 96:128].set(mate_table[0:2])
    small = small.at[55:56, 96:128].set(mate_table[3:4])

    gated = pl.pallas_call(
        _gated_body,
        out_shape=jax.ShapeDtypeStruct((NCOMBO, 128), f32),
    )(small, gate_W, gate_b.reshape(1, 128), feat_W, feat_b.reshape(1, 128))

    tfull = pl.pallas_call(
        _expand_body,
        grid=(17,),
        in_specs=[
            pl.BlockSpec((NCOMBO, 128), lambda i: (0, 0)),
            pl.BlockSpec((1, 1, 128), lambda i: (i, 0, 0)),
        ],
        out_specs=pl.BlockSpec((NCOMBO, 128), lambda i: (i, 0)),
        out_shape=jax.ShapeDtypeStruct((NROWS, 128), f32),
    )(gated, nuc_table.reshape(17, 1, 128))

    shp = (IDX_R, IDX_C)
    idx_args = [a.reshape(shp) for a in (nucleotide_sequences, cigar_encodings,
                                         base_qualities, strand_flags,
                                         mate_pair_flags)]
    ridx = pl.pallas_call(
        _index_body,
        grid=(IDX_R // IDX_BR,),
        in_specs=[pl.BlockSpec((IDX_BR, IDX_C), lambda i: (i, 0))] * 5,
        out_specs=pl.BlockSpec((IDX_BR, IDX_C), lambda i: (i, 0)),
        out_shape=jax.ShapeDtypeStruct(shp, jnp.int32),
    )(*idx_args)

    out = _gather_k(tfull, ridx.reshape(N))
    return out.reshape(B, S, D)


# speedup vs baseline: 40.8630x; 1.0002x over previous
import functools

import jax
import jax.numpy as jnp
from jax import lax
from jax.experimental import pallas as pl
from jax.experimental.pallas import tpu as pltpu
from jax.experimental.pallas import tpu_sc as plsc

B, S, D = 4096, 200, 128
N = B * S
NCOMBO = 7 * 41 * 16
NROWS = 17 * NCOMBO
NW = 32
PER_W = N // NW
CHUNK = 128
NCHUNK = PER_W // CHUNK

IDX_R, IDX_C = 3200, 256
IDX_BR = 400


def _gated_body(small_ref, gw_ref, gb_ref, fw_ref, fb_ref, out_ref):
    small = small_ref[...]
    tg = jnp.dot(small, gw_ref[...], preferred_element_type=jnp.float32)
    tf = jnp.dot(small, fw_ref[...], preferred_element_type=jnp.float32)
    r = lax.broadcasted_iota(jnp.int32, (NCOMBO, 64), 0)
    col = lax.broadcasted_iota(jnp.int32, (NCOMBO, 64), 1)
    sel = (
        (col == r // 656)
        | (col == 7 + (r // 16) % 41)
        | (col == 48 + (r % 16) // 4)
        | (col == 52 + r % 4)
    )
    e = sel.astype(jnp.float32)
    g = jnp.dot(e, tg, preferred_element_type=jnp.float32) + gb_ref[...]
    f = jnp.dot(e, tf, preferred_element_type=jnp.float32) + fb_ref[...]
    out_ref[...] = g * (1.0 / (1.0 + jnp.exp(-g))) * f


def _expand_body(gated_ref, nuc_ref, out_ref):
    out_ref[...] = gated_ref[...] + nuc_ref[0]


def _index_body(n_ref, c_ref, q_ref, s_ref, m_ref, out_ref):
    q = jnp.clip(q_ref[...], 0, 40)
    combo = (c_ref[...] * 41 + q) * 16 + s_ref[...] * 4 + m_ref[...]
    out_ref[...] = n_ref[...] * NCOMBO + combo


_SC_MESH = plsc.VectorSubcoreMesh(core_axis_name="c", subcore_axis_name="s")

NBUF = 5
LAG = 3


@functools.partial(
    pl.kernel,
    mesh=_SC_MESH,
    out_type=jax.ShapeDtypeStruct((N, D), jnp.float32),
    scratch_types=[
        pltpu.VMEM((NBUF, CHUNK), jnp.int32),
        pltpu.VMEM((NBUF, CHUNK, D), jnp.float32),
        pltpu.SemaphoreType.DMA((NBUF,)),
        pltpu.SemaphoreType.DMA((NBUF,)),
        pltpu.SemaphoreType.DMA((NBUF,)),
    ],
)
def _gather_k(table_hbm, idx_hbm, out_hbm, idx_v, rows_v, si, sg, sw):
    wid = lax.axis_index("s") * 2 + lax.axis_index("c")
    base = wid * PER_W

    for b in range(NBUF):
        pltpu.async_copy(idx_hbm.at[pl.ds(base + b * CHUNK, CHUNK)],
                         idx_v.at[b], si.at[b])

    @pl.loop(0, NCHUNK, step=NBUF)
    def _(g):
        for b in range(NBUF):
            m = g + b
            off = base + m * CHUNK
            bq = (b - LAG) % NBUF
            pltpu.make_async_copy(idx_hbm.at[pl.ds(off, CHUNK)],
                                  idx_v.at[b], si.at[b]).wait()

            @pl.when(m >= NBUF)
            def _():
                pltpu.make_async_copy(rows_v.at[b],
                                      out_hbm.at[pl.ds(base, CHUNK)],
                                      sw.at[b]).wait()

            pltpu.async_copy(table_hbm.at[idx_v.at[b]], rows_v.at[b],
                             sg.at[b])

            @pl.when(m >= LAG)
            def _():
                pltpu.make_async_copy(table_hbm.at[idx_v.at[bq]],
                                      rows_v.at[bq], sg.at[bq]).wait()
                pltpu.async_copy(rows_v.at[bq],
                                 out_hbm.at[pl.ds(off - LAG * CHUNK, CHUNK)],
                                 sw.at[bq])

            @pl.when((m >= LAG) & (m - LAG + NBUF < NCHUNK))
            def _():
                pltpu.async_copy(
                    idx_hbm.at[pl.ds(off + (NBUF - LAG) * CHUNK, CHUNK)],
                    idx_v.at[bq], si.at[bq])

    for j in range(LAG):
        mm = NCHUNK - LAG + j
        bb = mm % NBUF
        pltpu.make_async_copy(table_hbm.at[idx_v.at[bb]], rows_v.at[bb],
                              sg.at[bb]).wait()
        pltpu.async_copy(rows_v.at[bb],
                         out_hbm.at[pl.ds(base + mm * CHUNK, CHUNK)],
                         sw.at[bb])
    for b in range(NBUF):
        pltpu.make_async_copy(rows_v.at[b], out_hbm.at[pl.ds(base, CHUNK)],
                              sw.at[b]).wait()


def kernel(nucleotide_sequences, cigar_encodings, base_qualities, strand_flags,
           mate_pair_flags, nuc_table, cigar_table, quality_table, strand_table,
           mate_table, gate_W, gate_b, feat_W, feat_b):
    f32 = jnp.float32

    small = jnp.zeros((64, 128), f32)
    small = small.at[0:7, 0:32].set(cigar_table)
    small = small.at[7:48, 32:64].set(quality_table[0:41])
    small = small.at[48:52, 64:96].set(strand_table)
    small = small.at[52:54, 96:128].set(mate_table[0:2])
    small = small.at[55:56, 96:128].set(mate_table[3:4])

    gated = pl.pallas_call(
        _gated_body,
        out_shape=jax.ShapeDtypeStruct((NCOMBO, 128), f32),
    )(small, gate_W, gate_b.reshape(1, 128), feat_W, feat_b.reshape(1, 128))

    tfull = pl.pallas_call(
        _expand_body,
        grid=(17,),
        in_specs=[
            pl.BlockSpec((NCOMBO, 128), lambda i: (0, 0)),
            pl.BlockSpec((1, 1, 128), lambda i: (i, 0, 0)),
        ],
        out_specs=pl.BlockSpec((NCOMBO, 128), lambda i: (i, 0)),
        out_shape=jax.ShapeDtypeStruct((NROWS, 128), f32),
    )(gated, nuc_table.reshape(17, 1, 128))

    shp = (IDX_R, IDX_C)
    idx_args = [a.reshape(shp) for a in (nucleotide_sequences, cigar_encodings,
                                         base_qualities, strand_flags,
                                         mate_pair_flags)]
    ridx = pl.pallas_call(
        _index_body,
        grid=(IDX_R // IDX_BR,),
        in_specs=[pl.BlockSpec((IDX_BR, IDX_C), lambda i: (i, 0))] * 5,
        out_specs=pl.BlockSpec((IDX_BR, IDX_C), lambda i: (i, 0)),
        out_shape=jax.ShapeDtypeStruct(shp, jnp.int32),
    )(*idx_args)

    out = _gather_k(tfull, ridx.reshape(N))
    return out.reshape(B, S, D)


# speedup vs baseline: 41.1461x; 1.0069x over previous
import functools

import jax
import jax.numpy as jnp
from jax import lax
from jax.experimental import pallas as pl
from jax.experimental.pallas import tpu as pltpu
from jax.experimental.pallas import tpu_sc as plsc

B, S, D = 4096, 200, 128
N = B * S
NCOMBO = 7 * 41 * 16
NROWS = 17 * NCOMBO
NW = 32
PER_W = N // NW
CHUNK = 128
NCHUNK = PER_W // CHUNK

IDX_R, IDX_C = 3200, 256
IDX_BR = 400


def _table_body(small_ref, gw_ref, gb_ref, fw_ref, fb_ref, nuc_ref, out_ref,
                gated_sc):
    @pl.when(pl.program_id(0) == 0)
    def _():
        small = small_ref[...]
        tg = jnp.dot(small, gw_ref[...], preferred_element_type=jnp.float32)
        tf = jnp.dot(small, fw_ref[...], preferred_element_type=jnp.float32)
        r = lax.broadcasted_iota(jnp.int32, (NCOMBO, 64), 0)
        col = lax.broadcasted_iota(jnp.int32, (NCOMBO, 64), 1)
        sel = (
            (col == r // 656)
            | (col == 7 + (r // 16) % 41)
            | (col == 48 + (r % 16) // 4)
            | (col == 52 + r % 4)
        )
        e = sel.astype(jnp.float32)
        g = jnp.dot(e, tg, preferred_element_type=jnp.float32) + gb_ref[...]
        f = jnp.dot(e, tf, preferred_element_type=jnp.float32) + fb_ref[...]
        gated_sc[...] = g * (1.0 / (1.0 + jnp.exp(-g))) * f

    out_ref[...] = gated_sc[...] + nuc_ref[0]


def _index_body(n_ref, c_ref, q_ref, s_ref, m_ref, out_ref):
    q = jnp.clip(q_ref[...], 0, 40)
    combo = (c_ref[...] * 41 + q) * 16 + s_ref[...] * 4 + m_ref[...]
    out_ref[...] = n_ref[...] * NCOMBO + combo


_SC_MESH = plsc.VectorSubcoreMesh(core_axis_name="c", subcore_axis_name="s")

NBUF = 5
LAG = 3


@functools.partial(
    pl.kernel,
    mesh=_SC_MESH,
    out_type=jax.ShapeDtypeStruct((N, D), jnp.float32),
    scratch_types=[
        pltpu.VMEM((NBUF, CHUNK), jnp.int32),
        pltpu.VMEM((NBUF, CHUNK, D), jnp.float32),
        pltpu.SemaphoreType.DMA((NBUF,)),
        pltpu.SemaphoreType.DMA((NBUF,)),
        pltpu.SemaphoreType.DMA((NBUF,)),
    ],
)
def _gather_k(table_hbm, idx_hbm, out_hbm, idx_v, rows_v, si, sg, sw):
    wid = lax.axis_index("s") * 2 + lax.axis_index("c")
    base = wid * PER_W

    for b in range(NBUF):
        pltpu.async_copy(idx_hbm.at[pl.ds(base + b * CHUNK, CHUNK)],
                         idx_v.at[b], si.at[b])

    @pl.loop(0, NCHUNK, step=NBUF)
    def _(g):
        for b in range(NBUF):
            m = g + b
            off = base + m * CHUNK
            bq = (b - LAG) % NBUF
            pltpu.make_async_copy(idx_hbm.at[pl.ds(off, CHUNK)],
                                  idx_v.at[b], si.at[b]).wait()

            @pl.when(m >= NBUF)
            def _():
                pltpu.make_async_copy(rows_v.at[b],
                                      out_hbm.at[pl.ds(base, CHUNK)],
                                      sw.at[b]).wait()

            pltpu.async_copy(table_hbm.at[idx_v.at[b]], rows_v.at[b],
                             sg.at[b])

            @pl.when(m >= LAG)
            def _():
                pltpu.make_async_copy(table_hbm.at[idx_v.at[bq]],
                                      rows_v.at[bq], sg.at[bq]).wait()
                pltpu.async_copy(rows_v.at[bq],
                                 out_hbm.at[pl.ds(off - LAG * CHUNK, CHUNK)],
                                 sw.at[bq])

            @pl.when((m >= LAG) & (m - LAG + NBUF < NCHUNK))
            def _():
                pltpu.async_copy(
                    idx_hbm.at[pl.ds(off + (NBUF - LAG) * CHUNK, CHUNK)],
                    idx_v.at[bq], si.at[bq])

    for j in range(LAG):
        mm = NCHUNK - LAG + j
        bb = mm % NBUF
        pltpu.make_async_copy(table_hbm.at[idx_v.at[bb]], rows_v.at[bb],
                              sg.at[bb]).wait()
        pltpu.async_copy(rows_v.at[bb],
                         out_hbm.at[pl.ds(base + mm * CHUNK, CHUNK)],
                         sw.at[bb])
    for b in range(NBUF):
        pltpu.make_async_copy(rows_v.at[b], out_hbm.at[pl.ds(base, CHUNK)],
                              sw.at[b]).wait()


def kernel(nucleotide_sequences, cigar_encodings, base_qualities, strand_flags,
           mate_pair_flags, nuc_table, cigar_table, quality_table, strand_table,
           mate_table, gate_W, gate_b, feat_W, feat_b):
    f32 = jnp.float32

    small = jnp.zeros((64, 128), f32)
    small = small.at[0:7, 0:32].set(cigar_table)
    small = small.at[7:48, 32:64].set(quality_table[0:41])
    small = small.at[48:52, 64:96].set(strand_table)
    small = small.at[52:54, 96:128].set(mate_table[0:2])
    small = small.at[55:56, 96:128].set(mate_table[3:4])

    tfull = pl.pallas_call(
        _table_body,
        grid=(17,),
        in_specs=[
            pl.BlockSpec((64, 128), lambda i: (0, 0)),
            pl.BlockSpec((128, 128), lambda i: (0, 0)),
            pl.BlockSpec((1, 128), lambda i: (0, 0)),
            pl.BlockSpec((128, 128), lambda i: (0, 0)),
            pl.BlockSpec((1, 128), lambda i: (0, 0)),
            pl.BlockSpec((1, 1, 128), lambda i: (i, 0, 0)),
        ],
        out_specs=pl.BlockSpec((NCOMBO, 128), lambda i: (i, 0)),
        out_shape=jax.ShapeDtypeStruct((NROWS, 128), f32),
        scratch_shapes=[pltpu.VMEM((NCOMBO, 128), f32)],
    )(small, gate_W, gate_b.reshape(1, 128), feat_W, feat_b.reshape(1, 128),
      nuc_table.reshape(17, 1, 128))

    shp = (IDX_R, IDX_C)
    idx_args = [a.reshape(shp) for a in (nucleotide_sequences, cigar_encodings,
                                         base_qualities, strand_flags,
                                         mate_pair_flags)]
    ridx = pl.pallas_call(
        _index_body,
        grid=(IDX_R // IDX_BR,),
        in_specs=[pl.BlockSpec((IDX_BR, IDX_C), lambda i: (i, 0))] * 5,
        out_specs=pl.BlockSpec((IDX_BR, IDX_C), lambda i: (i, 0)),
        out_shape=jax.ShapeDtypeStruct(shp, jnp.int32),
    )(*idx_args)

    out = _gather_k(tfull, ridx.reshape(N))
    return out.reshape(B, S, D)


# speedup vs baseline: 43.8921x; 1.0667x over previous
import functools

import jax
import jax.numpy as jnp
from jax import lax
from jax.experimental import pallas as pl
from jax.experimental.pallas import tpu as pltpu
from jax.experimental.pallas import tpu_sc as plsc

B, S, D = 4096, 200, 128
N = B * S
NCOMBO = 7 * 41 * 16
NROWS = 17 * NCOMBO
NW = 32
PER_W = N // NW
CHUNK = 128
NCHUNK = PER_W // CHUNK

IDX_BR = 512


def _table_body(cig_ref, qual_ref, str_ref, mate_ref, gw_ref, gb_ref, fw_ref,
                fb_ref, nuc_ref, out_ref, gated_sc):
    @pl.when(pl.program_id(0) == 0)
    def _():
        mrow = lax.broadcasted_iota(jnp.int32, (4, 32), 0)
        mate_m = jnp.where(mrow == 2, 0.0, mate_ref[...])

        def fold(w_ref):
            w = w_ref[...]
            return jnp.concatenate([
                jnp.dot(cig_ref[...], w[0:32], preferred_element_type=jnp.float32),
                jnp.dot(qual_ref[0:41], w[32:64], preferred_element_type=jnp.float32),
                jnp.dot(str_ref[...], w[64:96], preferred_element_type=jnp.float32),
                jnp.dot(mate_m, w[96:128], preferred_element_type=jnp.float32),
            ], axis=0)

        tg = fold(gw_ref)
        tf = fold(fw_ref)
        r = lax.broadcasted_iota(jnp.int32, (NCOMBO, 56), 0)
        col = lax.broadcasted_iota(jnp.int32, (NCOMBO, 56), 1)
        sel = (
            (col == r // 656)
            | (col == 7 + (r // 16) % 41)
            | (col == 48 + (r % 16) // 4)
            | (col == 52 + r % 4)
        )
        e = sel.astype(jnp.float32)
        g = jnp.dot(e, tg, preferred_element_type=jnp.float32) + gb_ref[...]
        f = jnp.dot(e, tf, preferred_element_type=jnp.float32) + fb_ref[...]
        gated_sc[...] = g * (1.0 / (1.0 + jnp.exp(-g))) * f

    out_ref[...] = gated_sc[...] + nuc_ref[0]


def _index_body(n_ref, c_ref, q_ref, s_ref, m_ref, out_ref):
    q = jnp.clip(q_ref[...], 0, 40)
    combo = (c_ref[...] * 41 + q) * 16 + s_ref[...] * 4 + m_ref[...]
    out_ref[...] = n_ref[...] * NCOMBO + combo


_SC_MESH = plsc.VectorSubcoreMesh(core_axis_name="c", subcore_axis_name="s")

NBUF = 5
LAG = 3


@functools.partial(
    pl.kernel,
    mesh=_SC_MESH,
    out_type=jax.ShapeDtypeStruct((N, D), jnp.float32),
    scratch_types=[
        pltpu.VMEM((NBUF, CHUNK), jnp.int32),
        pltpu.VMEM((NBUF, CHUNK, D), jnp.float32),
        pltpu.SemaphoreType.DMA((NBUF,)),
        pltpu.SemaphoreType.DMA((NBUF,)),
        pltpu.SemaphoreType.DMA((NBUF,)),
    ],
)
def _gather_k(table_hbm, idx_hbm, out_hbm, idx_v, rows_v, si, sg, sw):
    wid = lax.axis_index("s") * 2 + lax.axis_index("c")
    base = wid * PER_W

    for b in range(NBUF):
        pltpu.async_copy(idx_hbm.at[pl.ds(base + b * CHUNK, CHUNK)],
                         idx_v.at[b], si.at[b])

    @pl.loop(0, NCHUNK, step=NBUF)
    def _(g):
        for b in range(NBUF):
            m = g + b
            off = base + m * CHUNK
            bq = (b - LAG) % NBUF
            pltpu.make_async_copy(idx_hbm.at[pl.ds(off, CHUNK)],
                                  idx_v.at[b], si.at[b]).wait()

            @pl.when(m >= NBUF)
            def _():
                pltpu.make_async_copy(rows_v.at[b],
                                      out_hbm.at[pl.ds(base, CHUNK)],
                                      sw.at[b]).wait()

            pltpu.async_copy(table_hbm.at[idx_v.at[b]], rows_v.at[b],
                             sg.at[b])

            @pl.when(m >= LAG)
            def _():
                pltpu.make_async_copy(table_hbm.at[idx_v.at[bq]],
                                      rows_v.at[bq], sg.at[bq]).wait()
                pltpu.async_copy(rows_v.at[bq],
                                 out_hbm.at[pl.ds(off - LAG * CHUNK, CHUNK)],
                                 sw.at[bq])

            @pl.when((m >= LAG) & (m - LAG + NBUF < NCHUNK))
            def _():
                pltpu.async_copy(
                    idx_hbm.at[pl.ds(off + (NBUF - LAG) * CHUNK, CHUNK)],
                    idx_v.at[bq], si.at[bq])

    for j in range(LAG):
        mm = NCHUNK - LAG + j
        bb = mm % NBUF
        pltpu.make_async_copy(table_hbm.at[idx_v.at[bb]], rows_v.at[bb],
                              sg.at[bb]).wait()
        pltpu.async_copy(rows_v.at[bb],
                         out_hbm.at[pl.ds(base + mm * CHUNK, CHUNK)],
                         sw.at[bb])
    for b in range(NBUF):
        pltpu.make_async_copy(rows_v.at[b], out_hbm.at[pl.ds(base, CHUNK)],
                              sw.at[b]).wait()


def kernel(nucleotide_sequences, cigar_encodings, base_qualities, strand_flags,
           mate_pair_flags, nuc_table, cigar_table, quality_table, strand_table,
           mate_table, gate_W, gate_b, feat_W, feat_b):
    f32 = jnp.float32

    tfull = pl.pallas_call(
        _table_body,
        grid=(17,),
        in_specs=[
            pl.BlockSpec((7, 32), lambda i: (0, 0)),
            pl.BlockSpec((43, 32), lambda i: (0, 0)),
            pl.BlockSpec((4, 32), lambda i: (0, 0)),
            pl.BlockSpec((4, 32), lambda i: (0, 0)),
            pl.BlockSpec((128, 128), lambda i: (0, 0)),
            pl.BlockSpec((1, 128), lambda i: (0, 0)),
            pl.BlockSpec((128, 128), lambda i: (0, 0)),
            pl.BlockSpec((1, 128), lambda i: (0, 0)),
            pl.BlockSpec((1, 1, 128), lambda i: (i, 0, 0)),
        ],
        out_specs=pl.BlockSpec((NCOMBO, 128), lambda i: (i, 0)),
        out_shape=jax.ShapeDtypeStruct((NROWS, 128), f32),
        scratch_shapes=[pltpu.VMEM((NCOMBO, 128), f32)],
    )(cigar_table, quality_table, strand_table, mate_table,
      gate_W, gate_b.reshape(1, 128), feat_W, feat_b.reshape(1, 128),
      nuc_table.reshape(17, 1, 128))

    ridx = pl.pallas_call(
        _index_body,
        grid=(B // IDX_BR,),
        in_specs=[pl.BlockSpec((IDX_BR, S), lambda i: (i, 0))] * 5,
        out_specs=pl.BlockSpec((IDX_BR, S), lambda i: (i, 0)),
        out_shape=jax.ShapeDtypeStruct((B, S), jnp.int32),
    )(nucleotide_sequences, cigar_encodings, base_qualities, strand_flags,
      mate_pair_flags)

    out = _gather_k(tfull, ridx.reshape(N))
    return out.reshape(B, S, D)


# speedup vs baseline: 46.4165x; 1.0575x over previous
import functools

import jax
import jax.numpy as jnp
from jax import lax
from jax.experimental import pallas as pl
from jax.experimental.pallas import tpu as pltpu
from jax.experimental.pallas import tpu_sc as plsc

B, S, D = 4096, 200, 128
N = B * S
NCOMBO = 7 * 41 * 16
NROWS = 17 * NCOMBO
NW = 32
PER_W = N // NW
CHUNK = 128
NCHUNK = PER_W // CHUNK

IDX_BR = 512


def _table_body(cig_ref, qual_ref, str_ref, mate_ref, gw_ref, gb_ref, fw_ref,
                fb_ref, nuc_ref, out_ref, gated_sc):
    @pl.when(pl.program_id(0) == 0)
    def _():
        mrow = lax.broadcasted_iota(jnp.int32, (4, 32), 0)
        mate_m = jnp.where(mrow == 2, 0.0, mate_ref[...])

        def fold(w_ref):
            w = w_ref[...]
            return jnp.concatenate([
                jnp.dot(cig_ref[...], w[0:32], preferred_element_type=jnp.float32),
                jnp.dot(qual_ref[0:41], w[32:64], preferred_element_type=jnp.float32),
                jnp.dot(str_ref[...], w[64:96], preferred_element_type=jnp.float32),
                jnp.dot(mate_m, w[96:128], preferred_element_type=jnp.float32),
            ], axis=0)

        tg = fold(gw_ref)
        tf = fold(fw_ref)
        r = lax.broadcasted_iota(jnp.int32, (NCOMBO, 56), 0)
        col = lax.broadcasted_iota(jnp.int32, (NCOMBO, 56), 1)
        sel = (
            (col == r // 656)
            | (col == 7 + (r // 16) % 41)
            | (col == 48 + (r % 16) // 4)
            | (col == 52 + r % 4)
        )
        e = sel.astype(jnp.float32)
        g = jnp.dot(e, tg, preferred_element_type=jnp.float32) + gb_ref[...]
        f = jnp.dot(e, tf, preferred_element_type=jnp.float32) + fb_ref[...]
        gated_sc[...] = g * (1.0 / (1.0 + jnp.exp(-g))) * f

    out_ref[...] = gated_sc[...] + nuc_ref[0]


def _unpack_chunk(p_row, out_row):
    for v in range(CHUNK // 16):
        sl = pl.ds(v * 16, 16)
        p = p_row[sl]
        n = p & 31
        c = (p >> 5) & 7
        q = jnp.minimum((p >> 8) & 63, 40)
        s = (p >> 14) & 3
        m = (p >> 16) & 3
        out_row[sl] = n * NCOMBO + c * 656 + q * 16 + s * 4 + m


_SC_MESH = plsc.VectorSubcoreMesh(core_axis_name="c", subcore_axis_name="s")

NBUF = 5
LAG = 3


@functools.partial(
    pl.kernel,
    mesh=_SC_MESH,
    out_type=jax.ShapeDtypeStruct((N, D), jnp.float32),
    scratch_types=[
        pltpu.VMEM((NBUF, CHUNK), jnp.int32),
        pltpu.VMEM((NBUF, CHUNK), jnp.int32),
        pltpu.VMEM((NBUF, CHUNK, D), jnp.float32),
        pltpu.SemaphoreType.DMA((NBUF,)),
        pltpu.SemaphoreType.DMA((NBUF,)),
        pltpu.SemaphoreType.DMA((NBUF,)),
    ],
)
def _gather_k(table_hbm, idx_hbm, out_hbm, pck_v, idx_v, rows_v, si, sg, sw):
    wid = lax.axis_index("s") * 2 + lax.axis_index("c")
    base = wid * PER_W

    for b in range(NBUF):
        pltpu.async_copy(idx_hbm.at[pl.ds(base + b * CHUNK, CHUNK)],
                         pck_v.at[b], si.at[b])

    @pl.loop(0, NCHUNK, step=NBUF)
    def _(g):
        for b in range(NBUF):
            m = g + b
            off = base + m * CHUNK
            bq = (b - LAG) % NBUF
            pltpu.make_async_copy(idx_hbm.at[pl.ds(off, CHUNK)],
                                  pck_v.at[b], si.at[b]).wait()
            _unpack_chunk(pck_v.at[b], idx_v.at[b])

            @pl.when(m >= NBUF)
            def _():
                pltpu.make_async_copy(rows_v.at[b],
                                      out_hbm.at[pl.ds(base, CHUNK)],
                                      sw.at[b]).wait()

            pltpu.async_copy(table_hbm.at[idx_v.at[b]], rows_v.at[b],
                             sg.at[b])

            @pl.when(m >= LAG)
            def _():
                pltpu.make_async_copy(table_hbm.at[idx_v.at[bq]],
                                      rows_v.at[bq], sg.at[bq]).wait()
                pltpu.async_copy(rows_v.at[bq],
                                 out_hbm.at[pl.ds(off - LAG * CHUNK, CHUNK)],
                                 sw.at[bq])

            @pl.when((m >= LAG) & (m - LAG + NBUF < NCHUNK))
            def _():
                pltpu.async_copy(
                    idx_hbm.at[pl.ds(off + (NBUF - LAG) * CHUNK, CHUNK)],
                    pck_v.at[bq], si.at[bq])

    for j in range(LAG):
        mm = NCHUNK - LAG + j
        bb = mm % NBUF
        pltpu.make_async_copy(table_hbm.at[idx_v.at[bb]], rows_v.at[bb],
                              sg.at[bb]).wait()
        pltpu.async_copy(rows_v.at[bb],
                         out_hbm.at[pl.ds(base + mm * CHUNK, CHUNK)],
                         sw.at[bb])
    for b in range(NBUF):
        pltpu.make_async_copy(rows_v.at[b], out_hbm.at[pl.ds(base, CHUNK)],
                              sw.at[b]).wait()


def kernel(nucleotide_sequences, cigar_encodings, base_qualities, strand_flags,
           mate_pair_flags, nuc_table, cigar_table, quality_table, strand_table,
           mate_table, gate_W, gate_b, feat_W, feat_b):
    f32 = jnp.float32

    tfull = pl.pallas_call(
        _table_body,
        grid=(17,),
        in_specs=[
            pl.BlockSpec((7, 32), lambda i: (0, 0)),
            pl.BlockSpec((43, 32), lambda i: (0, 0)),
            pl.BlockSpec((4, 32), lambda i: (0, 0)),
            pl.BlockSpec((4, 32), lambda i: (0, 0)),
            pl.BlockSpec((128, 128), lambda i: (0, 0)),
            pl.BlockSpec((1, 128), lambda i: (0, 0)),
            pl.BlockSpec((128, 128), lambda i: (0, 0)),
            pl.BlockSpec((1, 128), lambda i: (0, 0)),
            pl.BlockSpec((1, 1, 128), lambda i: (i, 0, 0)),
        ],
        out_specs=pl.BlockSpec((NCOMBO, 128), lambda i: (i, 0)),
        out_shape=jax.ShapeDtypeStruct((NROWS, 128), f32),
        scratch_shapes=[pltpu.VMEM((NCOMBO, 128), f32)],
    )(cigar_table, quality_table, strand_table, mate_table,
      gate_W, gate_b.reshape(1, 128), feat_W, feat_b.reshape(1, 128),
      nuc_table.reshape(17, 1, 128))

    packed = (nucleotide_sequences | (cigar_encodings << 5)
              | (base_qualities << 8) | (strand_flags << 14)
              | (mate_pair_flags << 16))

    out = _gather_k(tfull, packed.reshape(N))
    return out.reshape(B, S, D)
